# baseline (speedup 1.0000x reference)
import functools

import jax
import jax.numpy as jnp
from jax import lax
from jax.experimental import pallas as pl
from jax.experimental.pallas import tpu as pltpu
from jax.experimental.pallas import tpu_sc as plsc

D = 64
NI = 4096
NJ = 200
B = NI * NJ
NW = 32
B_PER_W = B // NW
C = 256
NT = C // 128
NB = 2
NCH = B_PER_W // C
NG = NCH // NB
TB_BYTES = 8 * NT * 8 * 128 * 4


@functools.partial(
    pl.kernel,
    mesh=plsc.VectorSubcoreMesh(core_axis_name="c", subcore_axis_name="s"),
    out_type=jax.ShapeDtypeStruct((NJ, 8, NI // 128, 8, 128), jnp.float32),
    compiler_params=pltpu.CompilerParams(
        use_tc_tiling_on_sc=False, needs_layout_passes=False),
    scratch_types=[
        pltpu.VMEM((C,), jnp.int32),
        pltpu.VMEM((C,), jnp.int32),
        pltpu.VMEM((C, D), jnp.float32),
        pltpu.VMEM((C, D), jnp.float32),
        pltpu.VMEM((8, NT, 8, 128), jnp.float32),
        pltpu.VMEM((8, NT, 8, 128), jnp.float32),
        pltpu.VMEM_SHARED((16, NB, 8, NT, 8, 128), jnp.float32),
        pltpu.SemaphoreType.DMA,
        pltpu.SemaphoreType.DMA,
        pltpu.SemaphoreType.DMA,
        pltpu.SemaphoreType.DMA,
        pltpu.SemaphoreType.DMA,
        pltpu.SemaphoreType.DMA,
    ],
)
def _gather_kernel(item_t_hbm, table_hbm, out_hbm,
                   idx0, idx1, rows0, rows1, tb0, tb1, sink_spmem,
                   si0, si1, sg0, sg1, so0, so1):
    wid = lax.axis_index("s") * 2 + lax.axis_index("c")
    base = wid * B_PER_W
    idxs = (idx0, idx1)
    rows = (rows0, rows1)
    tbs = (tb0, tb1)
    sis = (si0, si1)
    sgs = (sg0, sg1)
    sos = (so0, so1)

    iota16 = lax.iota(jnp.int32, 16)
    rivs = [[iota16 + (itl * 128 + g * 16) for g in range(8)]
            for itl in range(NT)]

    def idx_start(k, b):
        kc = jnp.minimum(k, B - C)
        return pltpu.make_async_copy(
            item_t_hbm.at[kc // NI, pl.ds(kc % NI, C)], idxs[b], sis[b])

    def out_copy(k, b):
        kc = jnp.maximum(k, 0)
        return pltpu.make_async_copy(
            tbs[b],
            out_hbm.at[kc // NI, :, pl.ds((kc % NI) // 128, NT), :, :],
            sos[b])

    def transpose_chunk(b):
        rv = rows[b]
        tb = tbs[b]

        def dt_body(dt, carry):
            base_col = dt * 8
            for itl in range(NT):
                for dr in range(8):
                    col = jnp.broadcast_to(base_col + dr, (16,)).astype(
                        jnp.int32)
                    for g in range(8):
                        vals = plsc.load_gather(rv, [rivs[itl][g], col])
                        tb[dt, itl, dr, pl.ds(g * 16, 16)] = vals
            return carry

        lax.fori_loop(0, 8, dt_body, 0, unroll=False)

    sid = lax.axis_index("s")
    for b in range(NB):
        pltpu.async_copy(tbs[b], sink_spmem.at[sid, b], sos[b])
        idx_start(base + b * C, b).start()

    def outer(g, carry):
        for b in range(NB):
            k = base + (g * NB + b) * C
            idx_start(k, b).wait()
            pltpu.async_copy(table_hbm.at[idxs[b]], rows[b], sgs[b]).wait()
            idx_start(k + NB * C, b).start()
            out_copy(k - NB * C, b).wait()
            transpose_chunk(b)
            out_copy(k, b).start()
        return carry

    lax.fori_loop(0, NG, outer, 0)

    for b in range(NB):
        idx_start(B - C, b).wait()
        out_copy(base + (NCH - NB + b) * C, b).wait()


def kernel(item, table):
    out5d = _gather_kernel(item.T, table)
    return out5d.transpose((2, 4, 0, 1, 3)).reshape(NI, NJ, D)


# speedup vs baseline: 1.7595x; 1.7595x over previous
import functools

import jax
import jax.numpy as jnp
from jax import lax
from jax.experimental import pallas as pl
from jax.experimental.pallas import tpu as pltpu
from jax.experimental.pallas import tpu_sc as plsc

D = 64
NI = 4096
NJ = 200
B = NI * NJ
NW = 32
B_PER_W = B // NW
C = 256
NT = C // 128
NB = 2
NCH = B_PER_W // C
NG = NCH // NB
TB_BYTES = 8 * NT * 8 * 128 * 4


@functools.partial(
    pl.kernel,
    mesh=plsc.VectorSubcoreMesh(core_axis_name="c", subcore_axis_name="s"),
    out_type=jax.ShapeDtypeStruct((NJ, 8, NI // 128, 8, 128), jnp.float32),
    compiler_params=pltpu.CompilerParams(
        use_tc_tiling_on_sc=False, needs_layout_passes=False),
    scratch_types=[
        pltpu.VMEM((C,), jnp.int32),
        pltpu.VMEM((C,), jnp.int32),
        pltpu.VMEM((C, D), jnp.float32),
        pltpu.VMEM((C, D), jnp.float32),
        pltpu.VMEM((8, 3, 8, 129), jnp.float32),
        pltpu.VMEM((8, 3, 8, 129), jnp.float32),
        pltpu.VMEM_SHARED((16, NB, C, D), jnp.float32),
        pltpu.SemaphoreType.DMA,
        pltpu.SemaphoreType.DMA,
        pltpu.SemaphoreType.DMA,
        pltpu.SemaphoreType.DMA,
        pltpu.SemaphoreType.DMA,
        pltpu.SemaphoreType.DMA,
    ],
)
def _gather_kernel(item_t_hbm, table_hbm, out_hbm,
                   idx0, idx1, rows0, rows1, tb0, tb1, sink_spmem,
                   si0, si1, sg0, sg1, so0, so1):
    wid = lax.axis_index("s") * 2 + lax.axis_index("c")
    base = wid * B_PER_W
    idxs = (idx0, idx1)
    rows = (rows0, rows1)
    tbs = (tb0, tb1)
    sis = (si0, si1)
    sgs = (sg0, sg1)
    sos = (so0, so1)

    iota16 = lax.iota(jnp.int32, 16)
    dt_vecs = [dg * 2 + iota16 // 8 for dg in range(D // 16)]
    dr_vec = iota16 % 8

    def idx_start(k, b):
        kc = jnp.minimum(k, B - C)
        return pltpu.make_async_copy(
            item_t_hbm.at[kc // NI, pl.ds(kc % NI, C)], idxs[b], sis[b])

    def out_wait(k, b):
        kc = jnp.maximum(k, 0)
        pltpu.make_async_copy(
            tbs[b].at[:, pl.ds(0, NT), :, pl.ds(0, 128)],
            out_hbm.at[kc // NI, :, pl.ds((kc % NI) // 128, NT), :, :],
            sos[b]).wait()

    def out_start(k, b):
        j = k // NI
        it0 = (k % NI) // 128
        for dt in range(8):
            for itl in range(NT):
                pltpu.make_async_copy(
                    tbs[b].at[dt, itl, :, pl.ds(0, 128)],
                    out_hbm.at[j, dt, it0 + itl, :, :],
                    sos[b]).start()

    def transpose_chunk(b):
        rv = rows[b]
        tb = tbs[b]

        def r_body(r, carry):
            itl_vec = jnp.broadcast_to(r // 128, (16,)).astype(jnp.int32)
            ir_vec = jnp.broadcast_to(r % 128, (16,)).astype(jnp.int32)
            for dg in range(D // 16):
                vals = rv[r, pl.ds(dg * 16, 16)]
                plsc.store_scatter(
                    tb, [dt_vecs[dg], itl_vec, dr_vec, ir_vec], vals)
            return carry

        lax.fori_loop(0, C, r_body, 0, unroll=False)

    sid = lax.axis_index("s")
    for b in range(NB):
        pltpu.async_copy(rows[b], sink_spmem.at[sid, b], sos[b])
        idx_start(base + b * C, b).start()

    def outer(g, carry):
        for b in range(NB):
            k = base + (g * NB + b) * C
            idx_start(k, b).wait()
            pltpu.async_copy(table_hbm.at[idxs[b]], rows[b], sgs[b]).wait()
            idx_start(k + NB * C, b).start()
            out_wait(k - NB * C, b)
            transpose_chunk(b)
            out_start(k, b)
        return carry

    lax.fori_loop(0, NG, outer, 0)

    for b in range(NB):
        idx_start(B - C, b).wait()
        out_wait(base + (NCH - NB + b) * C, b)


def kernel(item, table):
    out5d = _gather_kernel(item.T, table)
    return out5d.transpose((2, 4, 0, 1, 3)).reshape(NI, NJ, D)


# speedup vs baseline: 1.7965x; 1.0210x over previous
import functools

import jax
import jax.numpy as jnp
from jax import lax
from jax.experimental import pallas as pl
from jax.experimental.pallas import tpu as pltpu
from jax.experimental.pallas import tpu_sc as plsc

D = 64
NI = 4096
NJ = 200
B = NI * NJ
NW = 32
B_PER_W = B // NW
C = 256
NT = C // 128
NB = 2
NCH = B_PER_W // C
NG = NCH // NB
TB_BYTES = 8 * NT * 8 * 128 * 4


@functools.partial(
    pl.kernel,
    mesh=plsc.VectorSubcoreMesh(core_axis_name="c", subcore_axis_name="s"),
    out_type=jax.ShapeDtypeStruct((NJ, 8, NI // 128, 8, 128), jnp.float32),
    compiler_params=pltpu.CompilerParams(
        use_tc_tiling_on_sc=False, needs_layout_passes=False),
    scratch_types=[
        pltpu.VMEM((C,), jnp.int32),
        pltpu.VMEM((C,), jnp.int32),
        pltpu.VMEM((C, D), jnp.float32),
        pltpu.VMEM((C, D), jnp.float32),
        pltpu.VMEM((8, 3, 8, 129), jnp.float32),
        pltpu.VMEM((8, 3, 8, 129), jnp.float32),
        pltpu.VMEM_SHARED((16, NB, C, D), jnp.float32),
        pltpu.SemaphoreType.DMA,
        pltpu.SemaphoreType.DMA,
        pltpu.SemaphoreType.DMA,
        pltpu.SemaphoreType.DMA,
        pltpu.SemaphoreType.DMA,
        pltpu.SemaphoreType.DMA,
    ],
)
def _gather_kernel(item_t_hbm, table_hbm, out_hbm,
                   idx0, idx1, rows0, rows1, tb0, tb1, sink_spmem,
                   si0, si1, sg0, sg1, so0, so1):
    wid = lax.axis_index("s") * 2 + lax.axis_index("c")
    base = wid * B_PER_W
    idxs = (idx0, idx1)
    rows = (rows0, rows1)
    tbs = (tb0, tb1)
    sis = (si0, si1)
    sgs = (sg0, sg1)
    sos = (so0, so1)

    iota16 = lax.iota(jnp.int32, 16)
    dt_vecs = [dg * 2 + iota16 // 8 for dg in range(D // 16)]
    dr_vec = iota16 % 8

    def idx_start(k, b):
        kc = jnp.minimum(k, B - C)
        return pltpu.make_async_copy(
            item_t_hbm.at[kc // NI, pl.ds(kc % NI, C)], idxs[b], sis[b])

    def out_wait(k, b):
        kc = jnp.maximum(k, 0)
        pltpu.make_async_copy(
            tbs[b].at[:, pl.ds(0, NT), :, pl.ds(0, 128)],
            out_hbm.at[kc // NI, :, pl.ds((kc % NI) // 128, NT), :, :],
            sos[b]).wait()

    def out_start(k, b):
        j = k // NI
        it0 = (k % NI) // 128
        for dt in range(8):
            for itl in range(NT):
                pltpu.make_async_copy(
                    tbs[b].at[dt, itl, :, pl.ds(0, 128)],
                    out_hbm.at[j, dt, it0 + itl, :, :],
                    sos[b]).start()

    def transpose_chunk(b):
        rv = rows[b]
        tb = tbs[b]

        def r_body(r0, carry):
            for u in range(8):
                r = r0 * 8 + u
                itl_vec = jnp.broadcast_to(r // 128, (16,)).astype(jnp.int32)
                ir_vec = jnp.broadcast_to(r % 128, (16,)).astype(jnp.int32)
                for dg in range(D // 16):
                    vals = rv[r, pl.ds(dg * 16, 16)]
                    plsc.store_scatter(
                        tb, [dt_vecs[dg], itl_vec, dr_vec, ir_vec], vals)
            return carry

        lax.fori_loop(0, C // 8, r_body, 0, unroll=False)

    sid = lax.axis_index("s")
    for b in range(NB):
        pltpu.async_copy(rows[b], sink_spmem.at[sid, b], sos[b])
        idx_start(base + b * C, b).start()

    def outer(g, carry):
        for b in range(NB):
            k = base + (g * NB + b) * C
            idx_start(k, b).wait()
            pltpu.async_copy(table_hbm.at[idxs[b]], rows[b], sgs[b]).wait()
            idx_start(k + NB * C, b).start()
            out_wait(k - NB * C, b)
            transpose_chunk(b)
            out_start(k, b)
        return carry

    lax.fori_loop(0, NG, outer, 0)

    for b in range(NB):
        idx_start(B - C, b).wait()
        out_wait(base + (NCH - NB + b) * C, b)


def kernel(item, table):
    out5d = _gather_kernel(item.T, table)
    return out5d.transpose((2, 4, 0, 1, 3)).reshape(NI, NJ, D)
